# Initial kernel scaffold; baseline (speedup 1.0000x reference)
import jax
import jax.numpy as jnp
from jax.experimental import pallas as pl


def kernel(y_val, W1, b1, W2, ln_scale, ln_bias, var_idx, num_var):
    raise NotImplementedError("write your pallas kernel here")


# speedup vs baseline: 3.6721x; 3.6721x over previous
import functools

import jax
import jax.numpy as jnp
from jax import lax
from jax.experimental import pallas as pl
from jax.experimental.pallas import tpu as pltpu
from jax.experimental.pallas import tpu_sc as plsc

N_VAL = 320000
NUM_VAR = 10000
NVP = 10240
HID = 128

NC = 2
NS = 16
NW = NC * NS

EPW = N_VAL // NW
SCB1 = 40
CHUNK1 = 200
NSUB1 = CHUNK1 // SCB1
NCHUNK1 = EPW // CHUNK1
SCB = 80
CHUNK = 400
NSUB = CHUNK // SCB
NCHUNK = EPW // CHUNK
RPS = NVP // NS

_MESH = plsc.VectorSubcoreMesh(
    core_axis_name="c", subcore_axis_name="s", num_cores=NC, num_subcores=NS)


@functools.partial(
    pl.kernel,
    out_type=[
        jax.ShapeDtypeStruct((NC, NVP, HID), jnp.float32),
        jax.ShapeDtypeStruct((NC * NVP,), jnp.float32),
    ],
    mesh=_MESH,
    scratch_types=[
        pltpu.VMEM((CHUNK1,), jnp.int32),
        pltpu.VMEM((CHUNK1, HID), jnp.float32),
        pltpu.VMEM((SCB1,), jnp.float32),
        pltpu.VMEM_SHARED((NVP, HID), jnp.float32),
        pltpu.VMEM_SHARED((NVP,), jnp.float32),
    ],
)
def _sc_scatter(y_hbm, idx_hbm, zrow0_hbm, cnt0_hbm, ones_hbm,
                zpart_hbm, cpart_hbm,
                idx_v, rows_v, ones_v, z_sh, c_sh):
    c = lax.axis_index("c")
    s = lax.axis_index("s")
    wid = c * NS + s

    pltpu.sync_copy(zrow0_hbm.at[pl.ds(s * RPS, RPS)], z_sh.at[pl.ds(s * RPS, RPS)])
    pltpu.sync_copy(cnt0_hbm.at[pl.ds(s * RPS, RPS)], c_sh.at[pl.ds(s * RPS, RPS)])
    pltpu.sync_copy(ones_hbm, ones_v)
    plsc.subcore_barrier()

    ebase = wid * EPW

    def body(ch, carry):
        off = ebase + ch * CHUNK1
        pltpu.sync_copy(idx_hbm.at[pl.ds(off, CHUNK1)], idx_v)
        pltpu.sync_copy(y_hbm.at[pl.ds(off, CHUNK1)], rows_v)
        for j in range(NSUB1):
            ix = idx_v.at[pl.ds(j * SCB1, SCB1)]
            pltpu.sync_copy(rows_v.at[pl.ds(j * SCB1, SCB1)],
                            z_sh.at[ix], add=True)
            pltpu.sync_copy(ones_v, c_sh.at[ix], add=True)
        return carry

    lax.fori_loop(0, NCHUNK1, body, 0)
    plsc.subcore_barrier()

    pltpu.sync_copy(z_sh.at[pl.ds(s * RPS, RPS)],
                    zpart_hbm.at[c, pl.ds(s * RPS, RPS)])
    pltpu.sync_copy(c_sh.at[pl.ds(s * RPS, RPS)],
                    cpart_hbm.at[pl.ds(c * NVP + s * RPS, RPS)])


_MLP_BLOCK = 512


def _mlp_body(zp_ref, cnt_ref, w1_ref, b1_ref, w2_ref, ls_ref, lb_ref, h_ref):
    z = zp_ref[0] + zp_ref[1]
    cnt = (cnt_ref[0] + cnt_ref[1]).reshape(_MLP_BLOCK, 1)
    zv = z / jnp.clip(cnt, 1.0, None)
    h = jnp.dot(zv, w1_ref[...], preferred_element_type=jnp.float32,
                precision=lax.Precision.HIGHEST) + b1_ref[...]
    h = jnp.maximum(h, 0.0)
    h = jnp.dot(h, w2_ref[...], preferred_element_type=jnp.float32,
                precision=lax.Precision.HIGHEST)
    mu = jnp.mean(h, axis=-1, keepdims=True)
    var = jnp.mean((h - mu) ** 2, axis=-1, keepdims=True)
    h_ref[...] = (h - mu) * lax.rsqrt(var + 1e-5) * ls_ref[...] + lb_ref[...]


def _mlp(zpart, cpart, W1, b1, W2, ln_scale, ln_bias):
    nblk = NVP // _MLP_BLOCK
    return pl.pallas_call(
        _mlp_body,
        grid=(nblk,),
        in_specs=[
            pl.BlockSpec((NC, _MLP_BLOCK, HID), lambda i: (0, i, 0)),
            pl.BlockSpec((NC, _MLP_BLOCK), lambda i: (0, i)),
            pl.BlockSpec((HID, HID), lambda i: (0, 0)),
            pl.BlockSpec((1, HID), lambda i: (0, 0)),
            pl.BlockSpec((HID, HID), lambda i: (0, 0)),
            pl.BlockSpec((1, HID), lambda i: (0, 0)),
            pl.BlockSpec((1, HID), lambda i: (0, 0)),
        ],
        out_specs=pl.BlockSpec((_MLP_BLOCK, HID), lambda i: (i, 0)),
        out_shape=jax.ShapeDtypeStruct((NVP, HID), jnp.float32),
    )(zpart, cpart.reshape(NC, NVP), W1, b1.reshape(1, HID), W2,
      ln_scale.reshape(1, HID), ln_bias.reshape(1, HID))


@functools.partial(
    pl.kernel,
    out_type=jax.ShapeDtypeStruct((N_VAL, HID), jnp.float32),
    mesh=_MESH,
    scratch_types=[
        pltpu.VMEM((CHUNK,), jnp.int32),
        pltpu.VMEM((CHUNK, HID), jnp.float32),
        pltpu.VMEM((CHUNK, HID), jnp.float32),
        pltpu.SemaphoreType.DMA,
    ],
)
def _sc_gather(y_hbm, idx_hbm, h_hbm, out_hbm, idx_v, g_v, y_v, sem):
    c = lax.axis_index("c")
    s = lax.axis_index("s")
    wid = c * NS + s
    ebase = wid * EPW

    def body(ch, carry):
        off = ebase + ch * CHUNK
        pltpu.sync_copy(idx_hbm.at[pl.ds(off, CHUNK)], idx_v)
        pltpu.sync_copy(y_hbm.at[pl.ds(off, CHUNK)], y_v)
        for j in range(NSUB):
            pltpu.async_copy(h_hbm.at[idx_v.at[pl.ds(j * SCB, SCB)]],
                             g_v.at[pl.ds(j * SCB, SCB)], sem).wait()

        def add_row(r, cc):
            for k in range(HID // 16):
                sl = pl.ds(k * 16, 16)
                y_v[r, sl] = y_v[r, sl] + g_v[r, sl]
            return cc

        lax.fori_loop(0, CHUNK, add_row, 0)
        pltpu.sync_copy(y_v, out_hbm.at[pl.ds(off, CHUNK)])
        return carry

    lax.fori_loop(0, NCHUNK, body, 0)


def kernel(y_val, W1, b1, W2, ln_scale, ln_bias, var_idx, num_var):
    del num_var
    idx = var_idx.astype(jnp.int32)
    zrow0 = jnp.zeros((NVP, HID), jnp.float32)
    cnt0 = jnp.zeros((NVP,), jnp.float32)
    ones = jnp.ones((SCB1,), jnp.float32)
    zpart, cpart = _sc_scatter(y_val, idx, zrow0, cnt0, ones)
    h = _mlp(zpart, cpart, W1, b1, W2, ln_scale, ln_bias)
    return _sc_gather(y_val, idx, h)


# speedup vs baseline: 5.4254x; 1.4775x over previous
import functools

import jax
import jax.numpy as jnp
from jax import lax
from jax.experimental import pallas as pl
from jax.experimental.pallas import tpu as pltpu
from jax.experimental.pallas import tpu_sc as plsc

N_VAL = 320000
NUM_VAR = 10000
NVP = 10240
HID = 128

NC = 2
NS = 16
NW = NC * NS

EPW = N_VAL // NW
RPS = NVP // NS

SCB1 = 80
CHUNK1 = 160
NFULL1 = EPW // CHUNK1
TAIL1 = EPW - NFULL1 * CHUNK1
NPAIR1 = NFULL1 // 2

SCB3 = 40
CHUNK3 = 200
NSUB3 = CHUNK3 // SCB3
NCHUNK3 = EPW // CHUNK3

_MESH = plsc.VectorSubcoreMesh(
    core_axis_name="c", subcore_axis_name="s", num_cores=NC, num_subcores=NS)


@functools.partial(
    pl.kernel,
    out_type=[
        jax.ShapeDtypeStruct((NC, NVP, HID), jnp.float32),
        jax.ShapeDtypeStruct((NC * NVP,), jnp.float32),
    ],
    mesh=_MESH,
    scratch_types=[
        pltpu.VMEM((CHUNK1,), jnp.int32),
        pltpu.VMEM((CHUNK1,), jnp.int32),
        pltpu.VMEM((CHUNK1, HID), jnp.float32),
        pltpu.VMEM((CHUNK1, HID), jnp.float32),
        pltpu.VMEM((SCB1,), jnp.float32),
        pltpu.VMEM_SHARED((NVP, HID), jnp.float32),
        pltpu.VMEM_SHARED((NVP,), jnp.float32),
        pltpu.SemaphoreType.DMA,
        pltpu.SemaphoreType.DMA,
    ],
)
def _sc_scatter(y_hbm, idx_hbm, zrow0_hbm, cnt0_hbm, ones_hbm,
                zpart_hbm, cpart_hbm,
                idx_a, idx_b, rows_a, rows_b, ones_v, z_sh, c_sh,
                sem_a, sem_b):
    c = lax.axis_index("c")
    s = lax.axis_index("s")
    wid = c * NS + s
    ebase = wid * EPW

    pltpu.sync_copy(zrow0_hbm.at[pl.ds(s * RPS, RPS)], z_sh.at[pl.ds(s * RPS, RPS)])
    pltpu.sync_copy(cnt0_hbm.at[pl.ds(s * RPS, RPS)], c_sh.at[pl.ds(s * RPS, RPS)])
    pltpu.sync_copy(ones_hbm, ones_v)
    plsc.subcore_barrier()

    def issue_in(k, n, idx_v, rows_v, sem):
        off = ebase + k * CHUNK1
        pltpu.async_copy(idx_hbm.at[pl.ds(off, n)], idx_v.at[pl.ds(0, n)], sem)
        pltpu.async_copy(y_hbm.at[pl.ds(off, n)], rows_v.at[pl.ds(0, n)], sem)

    def wait_in(k, n, idx_v, rows_v, sem):
        off = ebase + k * CHUNK1
        pltpu.make_async_copy(
            y_hbm.at[pl.ds(off, n)], rows_v.at[pl.ds(0, n)], sem).wait()
        pltpu.make_async_copy(
            idx_hbm.at[pl.ds(off, n)], idx_v.at[pl.ds(0, n)], sem).wait()

    def scatter(n, idx_v, rows_v):
        for j in range(n // SCB1):
            ix = idx_v.at[pl.ds(j * SCB1, SCB1)]
            pltpu.sync_copy(rows_v.at[pl.ds(j * SCB1, SCB1)],
                            z_sh.at[ix], add=True)
            pltpu.sync_copy(ones_v, c_sh.at[ix], add=True)

    issue_in(0, CHUNK1, idx_a, rows_a, sem_a)

    def pair(i, carry):
        k = 2 * i
        wait_in(k, CHUNK1, idx_a, rows_a, sem_a)
        issue_in(k + 1, CHUNK1, idx_b, rows_b, sem_b)
        scatter(CHUNK1, idx_a, rows_a)
        wait_in(k + 1, CHUNK1, idx_b, rows_b, sem_b)
        issue_in(k + 2, CHUNK1, idx_a, rows_a, sem_a)
        scatter(CHUNK1, idx_b, rows_b)
        return carry

    lax.fori_loop(0, NPAIR1 - 1, pair, 0)
    wait_in(NFULL1 - 2, CHUNK1, idx_a, rows_a, sem_a)
    issue_in(NFULL1 - 1, CHUNK1, idx_b, rows_b, sem_b)
    scatter(CHUNK1, idx_a, rows_a)
    wait_in(NFULL1 - 1, CHUNK1, idx_b, rows_b, sem_b)
    issue_in(NFULL1, TAIL1, idx_a, rows_a, sem_a)
    scatter(CHUNK1, idx_b, rows_b)
    wait_in(NFULL1, TAIL1, idx_a, rows_a, sem_a)
    scatter(TAIL1, idx_a, rows_a)

    plsc.subcore_barrier()

    pltpu.sync_copy(z_sh.at[pl.ds(s * RPS, RPS)],
                    zpart_hbm.at[c, pl.ds(s * RPS, RPS)])
    pltpu.sync_copy(c_sh.at[pl.ds(s * RPS, RPS)],
                    cpart_hbm.at[pl.ds(c * NVP + s * RPS, RPS)])


_MLP_BLOCK = 512


def _mlp_body(zp_ref, cnt_ref, w1_ref, b1_ref, w2_ref, ls_ref, lb_ref, h_ref):
    z = zp_ref[0] + zp_ref[1]
    cnt = (cnt_ref[0] + cnt_ref[1]).reshape(_MLP_BLOCK, 1)
    zv = z / jnp.clip(cnt, 1.0, None)
    h = jnp.dot(zv, w1_ref[...], preferred_element_type=jnp.float32,
                precision=lax.Precision.HIGHEST) + b1_ref[...]
    h = jnp.maximum(h, 0.0)
    h = jnp.dot(h, w2_ref[...], preferred_element_type=jnp.float32,
                precision=lax.Precision.HIGHEST)
    mu = jnp.mean(h, axis=-1, keepdims=True)
    var = jnp.mean((h - mu) ** 2, axis=-1, keepdims=True)
    h_ref[...] = (h - mu) * lax.rsqrt(var + 1e-5) * ls_ref[...] + lb_ref[...]


def _mlp(zpart, cpart, W1, b1, W2, ln_scale, ln_bias):
    nblk = NVP // _MLP_BLOCK
    return pl.pallas_call(
        _mlp_body,
        grid=(nblk,),
        in_specs=[
            pl.BlockSpec((NC, _MLP_BLOCK, HID), lambda i: (0, i, 0)),
            pl.BlockSpec((NC, _MLP_BLOCK), lambda i: (0, i)),
            pl.BlockSpec((HID, HID), lambda i: (0, 0)),
            pl.BlockSpec((1, HID), lambda i: (0, 0)),
            pl.BlockSpec((HID, HID), lambda i: (0, 0)),
            pl.BlockSpec((1, HID), lambda i: (0, 0)),
            pl.BlockSpec((1, HID), lambda i: (0, 0)),
        ],
        out_specs=pl.BlockSpec((_MLP_BLOCK, HID), lambda i: (i, 0)),
        out_shape=jax.ShapeDtypeStruct((NVP, HID), jnp.float32),
    )(zpart, cpart.reshape(NC, NVP), W1, b1.reshape(1, HID), W2,
      ln_scale.reshape(1, HID), ln_bias.reshape(1, HID))


@functools.partial(
    pl.kernel,
    out_type=jax.ShapeDtypeStruct((N_VAL, HID), jnp.float32),
    mesh=_MESH,
    scratch_types=[
        pltpu.VMEM((CHUNK3,), jnp.int32),
        pltpu.VMEM((CHUNK3,), jnp.int32),
        pltpu.VMEM((CHUNK3,), jnp.int32),
        pltpu.VMEM((CHUNK3, HID), jnp.float32),
        pltpu.VMEM((CHUNK3, HID), jnp.float32),
        pltpu.VMEM((CHUNK3, HID), jnp.float32),
        pltpu.VMEM((CHUNK3, HID), jnp.float32),
        pltpu.SemaphoreType.DMA,
        pltpu.SemaphoreType.DMA,
        pltpu.SemaphoreType.DMA,
        pltpu.SemaphoreType.DMA,
        pltpu.SemaphoreType.DMA,
        pltpu.SemaphoreType.DMA,
        pltpu.SemaphoreType.DMA,
    ],
)
def _sc_gather(y_hbm, idx_hbm, h_hbm, out_hbm,
               idx0, idx1, idx2, y0, y1, y2, g_v,
               si0, si1, si2, so0, so1, so2, sg):
    c = lax.axis_index("c")
    s = lax.axis_index("s")
    wid = c * NS + s
    ebase = wid * EPW

    idx_s = (idx0, idx1, idx2)
    y_s = (y0, y1, y2)
    si_s = (si0, si1, si2)
    so_s = (so0, so1, so2)

    def issue_in(k, sl):
        off = ebase + k * CHUNK3
        pltpu.async_copy(idx_hbm.at[pl.ds(off, CHUNK3)], idx_s[sl], si_s[sl])
        pltpu.async_copy(y_hbm.at[pl.ds(off, CHUNK3)], y_s[sl], si_s[sl])

    def wait_in(k, sl):
        off = ebase + k * CHUNK3
        pltpu.make_async_copy(y_hbm.at[pl.ds(off, CHUNK3)], y_s[sl], si_s[sl]).wait()
        pltpu.make_async_copy(idx_hbm.at[pl.ds(off, CHUNK3)], idx_s[sl], si_s[sl]).wait()

    def wait_out(k, sl):
        off = ebase + k * CHUNK3
        pltpu.make_async_copy(y_s[sl], out_hbm.at[pl.ds(off, CHUNK3)], so_s[sl]).wait()

    def process(k, sl, wait_prev_out, issue_next_in):
        wait_in(k, sl)
        idx_v, y_v = idx_s[sl], y_s[sl]
        descs = []
        for j in range(NSUB3):
            descs.append(pltpu.async_copy(
                h_hbm.at[idx_v.at[pl.ds(j * SCB3, SCB3)]],
                g_v.at[pl.ds(j * SCB3, SCB3)], sg))
        for d in descs:
            d.wait()

        @plsc.parallel_loop(0, CHUNK3, 1, unroll=2)
        def add_row(r):
            for kk in range(HID // 16):
                slc = pl.ds(kk * 16, 16)
                y_v[r, slc] = y_v[r, slc] + g_v[r, slc]

        off = ebase + k * CHUNK3
        pltpu.async_copy(y_v, out_hbm.at[pl.ds(off, CHUNK3)], so_s[sl])
        if wait_prev_out:
            wait_out(k - 1, (sl + 2) % 3)
        if issue_next_in:
            issue_in(k + 2, (sl + 2) % 3)

    issue_in(0, 0)
    issue_in(1, 1)
    process(0, 0, False, True)
    process(1, 1, True, True)
    process(2, 2, True, True)

    def body(j, carry):
        k = 3 * j
        process(k, 0, True, True)
        process(k + 1, 1, True, True)
        process(k + 2, 2, True, True)
        return carry

    lax.fori_loop(1, (NCHUNK3 - 2) // 3, body, 0)
    process(NCHUNK3 - 2, 0, True, False)
    process(NCHUNK3 - 1, 1, True, False)
    wait_out(NCHUNK3 - 1, 1)


def kernel(y_val, W1, b1, W2, ln_scale, ln_bias, var_idx, num_var):
    del num_var
    idx = var_idx.astype(jnp.int32)
    zrow0 = jnp.zeros((NVP, HID), jnp.float32)
    cnt0 = jnp.zeros((NVP,), jnp.float32)
    ones = jnp.ones((SCB1,), jnp.float32)
    zpart, cpart = _sc_scatter(y_val, idx, zrow0, cnt0, ones)
    h = _mlp(zpart, cpart, W1, b1, W2, ln_scale, ln_bias)
    return _sc_gather(y_val, idx, h)


# speedup vs baseline: 5.4355x; 1.0019x over previous
import functools

import jax
import jax.numpy as jnp
from jax import lax
from jax.experimental import pallas as pl
from jax.experimental.pallas import tpu as pltpu
from jax.experimental.pallas import tpu_sc as plsc

N_VAL = 320000
NUM_VAR = 10000
NVP = 10240
HID = 128

NC = 2
NS = 16
NW = NC * NS

EPW = N_VAL // NW
RPS = NVP // NS

SCB1 = 80
CHUNK1 = 160
NFULL1 = EPW // CHUNK1
TAIL1 = EPW - NFULL1 * CHUNK1
NPAIR1 = NFULL1 // 2

G_SPLITS = ((0, 104), (104, 96))
CHUNK3 = 200
NCHUNK3 = EPW // CHUNK3

_MESH = plsc.VectorSubcoreMesh(
    core_axis_name="c", subcore_axis_name="s", num_cores=NC, num_subcores=NS)


@functools.partial(
    pl.kernel,
    out_type=[
        jax.ShapeDtypeStruct((NC, NVP, HID), jnp.float32),
        jax.ShapeDtypeStruct((NC * NVP,), jnp.float32),
    ],
    mesh=_MESH,
    scratch_types=[
        pltpu.VMEM((CHUNK1,), jnp.int32),
        pltpu.VMEM((CHUNK1,), jnp.int32),
        pltpu.VMEM((CHUNK1, HID), jnp.float32),
        pltpu.VMEM((CHUNK1, HID), jnp.float32),
        pltpu.VMEM((SCB1,), jnp.float32),
        pltpu.VMEM_SHARED((NVP, HID), jnp.float32),
        pltpu.VMEM_SHARED((NVP,), jnp.float32),
        pltpu.SemaphoreType.DMA,
        pltpu.SemaphoreType.DMA,
    ],
)
def _sc_scatter(y_hbm, idx_hbm, zrow0_hbm, cnt0_hbm, ones_hbm,
                zpart_hbm, cpart_hbm,
                idx_a, idx_b, rows_a, rows_b, ones_v, z_sh, c_sh,
                sem_a, sem_b):
    c = lax.axis_index("c")
    s = lax.axis_index("s")
    wid = c * NS + s
    ebase = wid * EPW

    pltpu.sync_copy(zrow0_hbm.at[pl.ds(s * RPS, RPS)], z_sh.at[pl.ds(s * RPS, RPS)])
    pltpu.sync_copy(cnt0_hbm.at[pl.ds(s * RPS, RPS)], c_sh.at[pl.ds(s * RPS, RPS)])
    pltpu.sync_copy(ones_hbm, ones_v)
    plsc.subcore_barrier()

    def issue_in(k, n, idx_v, rows_v, sem):
        off = ebase + k * CHUNK1
        pltpu.async_copy(idx_hbm.at[pl.ds(off, n)], idx_v.at[pl.ds(0, n)], sem)
        pltpu.async_copy(y_hbm.at[pl.ds(off, n)], rows_v.at[pl.ds(0, n)], sem)

    def wait_in(k, n, idx_v, rows_v, sem):
        off = ebase + k * CHUNK1
        pltpu.make_async_copy(
            y_hbm.at[pl.ds(off, n)], rows_v.at[pl.ds(0, n)], sem).wait()
        pltpu.make_async_copy(
            idx_hbm.at[pl.ds(off, n)], idx_v.at[pl.ds(0, n)], sem).wait()

    def scatter(n, idx_v, rows_v):
        for j in range(n // SCB1):
            ix = idx_v.at[pl.ds(j * SCB1, SCB1)]
            pltpu.sync_copy(rows_v.at[pl.ds(j * SCB1, SCB1)],
                            z_sh.at[ix], add=True)
            pltpu.sync_copy(ones_v, c_sh.at[ix], add=True)

    issue_in(0, CHUNK1, idx_a, rows_a, sem_a)

    def pair(i, carry):
        k = 2 * i
        wait_in(k, CHUNK1, idx_a, rows_a, sem_a)
        issue_in(k + 1, CHUNK1, idx_b, rows_b, sem_b)
        scatter(CHUNK1, idx_a, rows_a)
        wait_in(k + 1, CHUNK1, idx_b, rows_b, sem_b)
        issue_in(k + 2, CHUNK1, idx_a, rows_a, sem_a)
        scatter(CHUNK1, idx_b, rows_b)
        return carry

    lax.fori_loop(0, NPAIR1 - 1, pair, 0)
    wait_in(NFULL1 - 2, CHUNK1, idx_a, rows_a, sem_a)
    issue_in(NFULL1 - 1, CHUNK1, idx_b, rows_b, sem_b)
    scatter(CHUNK1, idx_a, rows_a)
    wait_in(NFULL1 - 1, CHUNK1, idx_b, rows_b, sem_b)
    issue_in(NFULL1, TAIL1, idx_a, rows_a, sem_a)
    scatter(CHUNK1, idx_b, rows_b)
    wait_in(NFULL1, TAIL1, idx_a, rows_a, sem_a)
    scatter(TAIL1, idx_a, rows_a)

    plsc.subcore_barrier()

    pltpu.sync_copy(z_sh.at[pl.ds(s * RPS, RPS)],
                    zpart_hbm.at[c, pl.ds(s * RPS, RPS)])
    pltpu.sync_copy(c_sh.at[pl.ds(s * RPS, RPS)],
                    cpart_hbm.at[pl.ds(c * NVP + s * RPS, RPS)])


_MLP_BLOCK = 512


def _mlp_body(zp_ref, cnt_ref, w1_ref, b1_ref, w2_ref, ls_ref, lb_ref, h_ref):
    z = zp_ref[0] + zp_ref[1]
    cnt = (cnt_ref[0] + cnt_ref[1]).reshape(_MLP_BLOCK, 1)
    zv = z / jnp.clip(cnt, 1.0, None)
    h = jnp.dot(zv, w1_ref[...], preferred_element_type=jnp.float32,
                precision=lax.Precision.HIGHEST) + b1_ref[...]
    h = jnp.maximum(h, 0.0)
    h = jnp.dot(h, w2_ref[...], preferred_element_type=jnp.float32,
                precision=lax.Precision.HIGHEST)
    mu = jnp.mean(h, axis=-1, keepdims=True)
    var = jnp.mean((h - mu) ** 2, axis=-1, keepdims=True)
    h_ref[...] = (h - mu) * lax.rsqrt(var + 1e-5) * ls_ref[...] + lb_ref[...]


def _mlp(zpart, cpart, W1, b1, W2, ln_scale, ln_bias):
    nblk = NVP // _MLP_BLOCK
    return pl.pallas_call(
        _mlp_body,
        grid=(nblk,),
        in_specs=[
            pl.BlockSpec((NC, _MLP_BLOCK, HID), lambda i: (0, i, 0)),
            pl.BlockSpec((NC, _MLP_BLOCK), lambda i: (0, i)),
            pl.BlockSpec((HID, HID), lambda i: (0, 0)),
            pl.BlockSpec((1, HID), lambda i: (0, 0)),
            pl.BlockSpec((HID, HID), lambda i: (0, 0)),
            pl.BlockSpec((1, HID), lambda i: (0, 0)),
            pl.BlockSpec((1, HID), lambda i: (0, 0)),
        ],
        out_specs=pl.BlockSpec((_MLP_BLOCK, HID), lambda i: (i, 0)),
        out_shape=jax.ShapeDtypeStruct((NVP, HID), jnp.float32),
    )(zpart, cpart.reshape(NC, NVP), W1, b1.reshape(1, HID), W2,
      ln_scale.reshape(1, HID), ln_bias.reshape(1, HID))


@functools.partial(
    pl.kernel,
    out_type=jax.ShapeDtypeStruct((N_VAL, HID), jnp.float32),
    mesh=_MESH,
    scratch_types=[
        pltpu.VMEM((CHUNK3,), jnp.int32),
        pltpu.VMEM((CHUNK3,), jnp.int32),
        pltpu.VMEM((CHUNK3,), jnp.int32),
        pltpu.VMEM((CHUNK3, HID), jnp.float32),
        pltpu.VMEM((CHUNK3, HID), jnp.float32),
        pltpu.VMEM((CHUNK3, HID), jnp.float32),
        pltpu.VMEM((CHUNK3, HID), jnp.float32),
        pltpu.SemaphoreType.DMA,
        pltpu.SemaphoreType.DMA,
        pltpu.SemaphoreType.DMA,
        pltpu.SemaphoreType.DMA,
        pltpu.SemaphoreType.DMA,
        pltpu.SemaphoreType.DMA,
        pltpu.SemaphoreType.DMA,
    ],
)
def _sc_gather(y_hbm, idx_hbm, h_hbm, out_hbm,
               idx0, idx1, idx2, y0, y1, y2, g_v,
               si0, si1, si2, so0, so1, so2, sg):
    c = lax.axis_index("c")
    s = lax.axis_index("s")
    wid = c * NS + s
    ebase = wid * EPW

    idx_s = (idx0, idx1, idx2)
    y_s = (y0, y1, y2)
    si_s = (si0, si1, si2)
    so_s = (so0, so1, so2)

    def issue_in(k, sl):
        off = ebase + k * CHUNK3
        pltpu.async_copy(idx_hbm.at[pl.ds(off, CHUNK3)], idx_s[sl], si_s[sl])
        pltpu.async_copy(y_hbm.at[pl.ds(off, CHUNK3)], y_s[sl], si_s[sl])

    def wait_in(k, sl):
        off = ebase + k * CHUNK3
        pltpu.make_async_copy(y_hbm.at[pl.ds(off, CHUNK3)], y_s[sl], si_s[sl]).wait()
        pltpu.make_async_copy(idx_hbm.at[pl.ds(off, CHUNK3)], idx_s[sl], si_s[sl]).wait()

    def wait_out(k, sl):
        off = ebase + k * CHUNK3
        pltpu.make_async_copy(y_s[sl], out_hbm.at[pl.ds(off, CHUNK3)], so_s[sl]).wait()

    def process(k, sl, wait_prev_out, issue_next_in):
        wait_in(k, sl)
        idx_v, y_v = idx_s[sl], y_s[sl]
        descs = []
        for (o, n) in G_SPLITS:
            descs.append(pltpu.async_copy(
                h_hbm.at[idx_v.at[pl.ds(o, n)]],
                g_v.at[pl.ds(o, n)], sg))
        for d in descs:
            d.wait()

        @plsc.parallel_loop(0, CHUNK3, 1, unroll=4)
        def add_row(r):
            for kk in range(HID // 16):
                slc = pl.ds(kk * 16, 16)
                y_v[r, slc] = y_v[r, slc] + g_v[r, slc]

        off = ebase + k * CHUNK3
        pltpu.async_copy(y_v, out_hbm.at[pl.ds(off, CHUNK3)], so_s[sl])
        if wait_prev_out:
            wait_out(k - 1, (sl + 2) % 3)
        if issue_next_in:
            issue_in(k + 2, (sl + 2) % 3)

    issue_in(0, 0)
    issue_in(1, 1)
    process(0, 0, False, True)
    process(1, 1, True, True)
    process(2, 2, True, True)

    def body(j, carry):
        k = 3 * j
        process(k, 0, True, True)
        process(k + 1, 1, True, True)
        process(k + 2, 2, True, True)
        return carry

    lax.fori_loop(1, (NCHUNK3 - 2) // 3, body, 0)
    process(NCHUNK3 - 2, 0, True, False)
    process(NCHUNK3 - 1, 1, True, False)
    wait_out(NCHUNK3 - 1, 1)


def kernel(y_val, W1, b1, W2, ln_scale, ln_bias, var_idx, num_var):
    del num_var
    idx = var_idx.astype(jnp.int32)
    zrow0 = jnp.zeros((NVP, HID), jnp.float32)
    cnt0 = jnp.zeros((NVP,), jnp.float32)
    ones = jnp.ones((SCB1,), jnp.float32)
    zpart, cpart = _sc_scatter(y_val, idx, zrow0, cnt0, ones)
    h = _mlp(zpart, cpart, W1, b1, W2, ln_scale, ln_bias)
    return _sc_gather(y_val, idx, h)
